# baseline (speedup 1.0000x reference)
import jax
import jax.numpy as jnp
from jax import lax
from jax.experimental import pallas as pl
from jax.experimental.pallas import tpu as pltpu
from jax.experimental.pallas import tpu_sc as plsc

B = 16
T = 2048
D = 64
VOCAB = 1000000
NC = 2
NS = 16
NW = NC * NS
NB = 8
TW = 128
LANES = 16
VPD = D // LANES
NCHUNK = TW // LANES


def _emb_body(x_hbm, wte_hbm, wpe2_hbm, out2_hbm,
              idx_v, slab_v, stage_v, sem_a, sem_b):
    wid = lax.axis_index("s") * NC + lax.axis_index("c")
    b0 = pl.multiple_of((wid % 2) * NB, NB)
    t0 = pl.multiple_of((wid // 2) * TW, TW)

    pltpu.sync_copy(x_hbm.at[pl.ds(b0, NB), pl.ds(t0, TW)], idx_v)

    sems = (sem_a, sem_b)

    def fire_chunk(b, c):
        par = c % 2
        vv = idx_v[b, pl.ds(c * LANES, LANES)]
        descs = []
        for i in range(LANES):
            ix = vv[i]
            slab8 = pl.multiple_of(lax.shift_right_logical(ix, 3) * 8, 8)
            descs.append(pltpu.async_copy(
                wte_hbm.at[pl.ds(slab8, 8), :],
                slab_v.at[par * LANES + i], sems[par]))
        return descs

    def run_b(b, carry):
        pltpu.sync_copy(
            wpe2_hbm.at[pl.ds(pl.multiple_of(t0 * D // 128, 64), TW * D // 128)],
            stage_v)

        descs = fire_chunk(b, 0)
        for c in range(NCHUNK):
            nxt = fire_chunk(b, c + 1) if c + 1 < NCHUNK else []
            for d in descs:
                d.wait()
            descs = nxt
            par = c % 2
            vv = idx_v[b, pl.ds(c * LANES, LANES)]
            for i in range(LANES):
                row = jnp.bitwise_and(vv[i], 7)
                jj = c * LANES + i
                r2 = jj // 2
                c2 = (jj % 2) * D
                for v in range(VPD):
                    val = slab_v[par * LANES + i, row, pl.ds(v * LANES, LANES)]
                    plsc.addupdate(
                        stage_v.at[r2, pl.ds(c2 + v * LANES, LANES)], val)

        out_off = pl.multiple_of(((b0 + b) * T + t0) * D // 128, 64)
        pltpu.sync_copy(stage_v, out2_hbm.at[pl.ds(out_off, TW * D // 128)])
        return carry

    lax.fori_loop(0, NB, run_b, 0)


@jax.jit
def kernel(x, wte, wpe):
    wpe2 = wpe.reshape(T * D // 128, 128)
    run = pl.kernel(
        _emb_body,
        out_type=jax.ShapeDtypeStruct((B * T * D // 128, 128), jnp.float32),
        mesh=plsc.VectorSubcoreMesh(core_axis_name="c", subcore_axis_name="s"),
        scratch_types=[
            pltpu.VMEM((NB, TW), jnp.int32),
            pltpu.VMEM((2 * LANES, 8, D), jnp.float32),
            pltpu.VMEM((TW * D // 128, 128), jnp.float32),
            pltpu.SemaphoreType.DMA,
            pltpu.SemaphoreType.DMA,
        ],
    )
    out2 = run(x, wte, wpe2)
    return out2.reshape(B, T, D)


# speedup vs baseline: 1.0026x; 1.0026x over previous
import jax
import jax.numpy as jnp
from jax import lax
from jax.experimental import pallas as pl
from jax.experimental.pallas import tpu as pltpu
from jax.experimental.pallas import tpu_sc as plsc

B = 16
T = 2048
D = 64
VOCAB = 1000000
NC = 2
NS = 16
NW = NC * NS
NB = 8
TW = 128
LANES = 16
VPD = D // LANES
NCHUNK = TW // LANES


def _emb_body(x_hbm, wte_hbm, wpe2_hbm, out2_hbm,
              idx_v, slab_v, stage_v, sem_a, sem_b):
    wid = lax.axis_index("s") * NC + lax.axis_index("c")
    b0 = pl.multiple_of((wid % 2) * NB, NB)
    t0 = pl.multiple_of((wid // 2) * TW, TW)

    pltpu.sync_copy(x_hbm.at[pl.ds(b0, NB), pl.ds(t0, TW)], idx_v)

    sems = (sem_a, sem_b)
    CH = 32

    def fire_chunk(b, c):
        par = c % 2
        descs = []
        for g in range(CH // LANES):
            vv = idx_v[b, pl.ds(c * CH + g * LANES, LANES)]
            for i in range(LANES):
                ix = vv[i]
                slab8 = pl.multiple_of(lax.shift_right_logical(ix, 3) * 8, 8)
                descs.append(pltpu.async_copy(
                    wte_hbm.at[pl.ds(slab8, 8), :],
                    slab_v.at[par * CH + g * LANES + i], sems[par]))
        return descs

    def run_b(b, carry):
        pltpu.sync_copy(
            wpe2_hbm.at[pl.ds(pl.multiple_of(t0 * D // 128, 64), TW * D // 128)],
            stage_v)

        descs = fire_chunk(b, 0)
        for c in range(TW // CH):
            nxt = fire_chunk(b, c + 1) if c + 1 < TW // CH else []
            for d in descs:
                d.wait()
            descs = nxt
            par = c % 2
            for g in range(CH // LANES):
                vv = idx_v[b, pl.ds(c * CH + g * LANES, LANES)]
                for i in range(LANES):
                    row = jnp.bitwise_and(vv[i], 7)
                    jj = c * CH + g * LANES + i
                    r2 = jj // 2
                    c2 = (jj % 2) * D
                    for v in range(VPD):
                        val = slab_v[par * CH + g * LANES + i, row,
                                     pl.ds(v * LANES, LANES)]
                        plsc.addupdate(
                            stage_v.at[r2, pl.ds(c2 + v * LANES, LANES)], val)

        out_off = pl.multiple_of(((b0 + b) * T + t0) * D // 128, 64)
        pltpu.sync_copy(stage_v, out2_hbm.at[pl.ds(out_off, TW * D // 128)])
        return carry

    lax.fori_loop(0, NB, run_b, 0)


@jax.jit
def kernel(x, wte, wpe):
    wpe2 = wpe.reshape(T * D // 128, 128)
    run = pl.kernel(
        _emb_body,
        out_type=jax.ShapeDtypeStruct((B * T * D // 128, 128), jnp.float32),
        mesh=plsc.VectorSubcoreMesh(core_axis_name="c", subcore_axis_name="s"),
        scratch_types=[
            pltpu.VMEM((NB, TW), jnp.int32),
            pltpu.VMEM((2 * 32, 8, D), jnp.float32),
            pltpu.VMEM((TW * D // 128, 128), jnp.float32),
            pltpu.SemaphoreType.DMA,
            pltpu.SemaphoreType.DMA,
        ],
    )
    out2 = run(x, wte, wpe2)
    return out2.reshape(B, T, D)
